# baseline (speedup 1.0000x reference)
import jax
import jax.numpy as jnp
from jax.experimental import pallas as pl
from jax.experimental.pallas import tpu as pltpu

import functools

import numpy as np

_E = 64
_K = 2
_BLK = 1024
_SUB = 256


@functools.lru_cache(maxsize=None)
def _tril_const(blk):
    return jnp.asarray(np.tril(np.ones((blk, blk), dtype=np.float32))
                       .astype(jnp.bfloat16))


def _pass1(x_ref, wg_ref, tril_ref, logits_ref, topk_ref, gates_ref, locp_ref,
           c0_ref, me_ref, run0, run1, me_acc):
    i = pl.program_id(0)

    @pl.when(i == 0)
    def _init():
        run0[...] = jnp.zeros_like(run0)
        run1[...] = jnp.zeros_like(run1)
        me_acc[...] = jnp.zeros_like(me_acc)

    logits = jax.lax.dot_general(
        x_ref[...], wg_ref[...], (((1,), (0,)), ((), ())),
        precision=jax.lax.Precision.DEFAULT,
        preferred_element_type=jnp.float32)
    logits_ref[...] = logits

    mx = jnp.max(logits, axis=1, keepdims=True)
    ex = jnp.exp(logits - mx)
    scores = ex / jnp.sum(ex, axis=1, keepdims=True)
    me_acc[...] += jnp.sum(scores, axis=0, keepdims=True)

    iota = jax.lax.broadcasted_iota(jnp.int32, scores.shape, 1)
    v0 = jnp.max(scores, axis=1, keepdims=True)
    i0 = jnp.min(jnp.where(scores == v0, iota, _E), axis=1, keepdims=True)
    m0b = iota == i0
    masked = jnp.where(m0b, -jnp.inf, scores)
    v1 = jnp.max(masked, axis=1, keepdims=True)
    i1 = jnp.min(jnp.where(masked == v1, iota, _E), axis=1, keepdims=True)
    m0 = m0b.astype(jnp.float32)
    m1 = (iota == i1).astype(jnp.float32)

    mcat = jnp.concatenate([m0, m1], axis=1).astype(jnp.bfloat16)
    tril = tril_ref[...]
    base = jnp.concatenate([run0[...], run1[...]], axis=1)
    locp = []
    for s in range(_BLK // _SUB):
        pref_s = jnp.dot(tril, mcat[s * _SUB:(s + 1) * _SUB, :],
                         preferred_element_type=jnp.float32)
        full_s = pref_s + (base - 1.0)
        m0_s = m0[s * _SUB:(s + 1) * _SUB, :]
        m1_s = m1[s * _SUB:(s + 1) * _SUB, :]
        loc0_s = jnp.sum(full_s[:, :_E] * m0_s, axis=1, keepdims=True)
        loc1_s = jnp.sum(full_s[:, _E:] * m1_s, axis=1, keepdims=True)
        locp.append(jnp.concatenate([loc0_s, loc1_s], axis=1))
        base = base + pref_s[_SUB - 1:_SUB, :]

    locp_ref[...] = jnp.concatenate(locp, axis=0).astype(jnp.int32)
    topk_ref[...] = jnp.concatenate([i0, i1], axis=1)
    den = jnp.maximum(v0 + v1, 1e-9)
    gates_ref[...] = jnp.concatenate([v0 / den, v1 / den], axis=1)

    run0[...] = base[:, :_E]
    run1[...] = base[:, _E:]

    @pl.when(i == pl.num_programs(0) - 1)
    def _fin():
        c0_ref[...] = run0[...]
        me_ref[...] = me_acc[...]


def _pass2(n_tokens, topk_ref, locp_ref, c0_ref, me_ref, loc_ref, laux_ref):
    i = pl.program_id(0)
    i1 = topk_ref[:, 1:2]
    iota = jax.lax.broadcasted_iota(jnp.int32, (_BLK, _E), 1)
    m1 = (iota == i1).astype(jnp.float32)
    add = jnp.sum(m1 * c0_ref[...], axis=1, keepdims=True)
    loc1 = locp_ref[:, 1:2] + add.astype(jnp.int32)
    loc_ref[...] = jnp.concatenate([locp_ref[:, 0:1], loc1], axis=1)

    @pl.when(i == 0)
    def _laux():
        scale = jnp.float32(_E) / jnp.float32(n_tokens * n_tokens)
        laux_ref[...] = (jnp.sum(me_ref[...] * c0_ref[...]) * scale
                         ).reshape(1, 1)


def kernel(x, wg, num_shards):
    n, d = x.shape
    nb = n // _BLK

    logits, topk_idx, gates, locp, c0, me_sum = pl.pallas_call(
        _pass1,
        grid=(nb,),
        in_specs=[
            pl.BlockSpec((_BLK, d), lambda i: (i, 0)),
            pl.BlockSpec((d, _E), lambda i: (0, 0)),
            pl.BlockSpec((_SUB, _SUB), lambda i: (0, 0)),
        ],
        out_specs=[
            pl.BlockSpec((_BLK, _E), lambda i: (i, 0)),
            pl.BlockSpec((_BLK, _K), lambda i: (i, 0)),
            pl.BlockSpec((_BLK, _K), lambda i: (i, 0)),
            pl.BlockSpec((_BLK, _K), lambda i: (i, 0)),
            pl.BlockSpec((1, _E), lambda i: (0, 0)),
            pl.BlockSpec((1, _E), lambda i: (0, 0)),
        ],
        out_shape=[
            jax.ShapeDtypeStruct((n, _E), jnp.float32),
            jax.ShapeDtypeStruct((n, _K), jnp.int32),
            jax.ShapeDtypeStruct((n, _K), jnp.float32),
            jax.ShapeDtypeStruct((n, _K), jnp.int32),
            jax.ShapeDtypeStruct((1, _E), jnp.float32),
            jax.ShapeDtypeStruct((1, _E), jnp.float32),
        ],
        scratch_shapes=[
            pltpu.VMEM((1, _E), jnp.float32),
            pltpu.VMEM((1, _E), jnp.float32),
            pltpu.VMEM((1, _E), jnp.float32),
        ],
    )(x, wg, _tril_const(_SUB))

    locations, laux = pl.pallas_call(
        lambda *refs: _pass2(n, *refs),
        grid=(nb,),
        in_specs=[
            pl.BlockSpec((_BLK, _K), lambda i: (i, 0)),
            pl.BlockSpec((_BLK, _K), lambda i: (i, 0)),
            pl.BlockSpec((1, _E), lambda i: (0, 0)),
            pl.BlockSpec((1, _E), lambda i: (0, 0)),
        ],
        out_specs=[
            pl.BlockSpec((_BLK, _K), lambda i: (i, 0)),
            pl.BlockSpec((1, 1), lambda i: (0, 0)),
        ],
        out_shape=[
            jax.ShapeDtypeStruct((n, _K), jnp.int32),
            jax.ShapeDtypeStruct((1, 1), jnp.float32),
        ],
    )(topk_idx, locp, c0, me_sum)

    l_aux = laux.reshape(())
    alignment = jnp.asarray(num_shards, dtype=jnp.int32) * 1
    capacity = _K * ((n + _E - 1) // _E)
    cap_arr = (((capacity + alignment - 1) // alignment) * alignment
               ).astype(jnp.int32)
    return (logits, l_aux, topk_idx, locations, gates, cap_arr)


# speedup vs baseline: 1.0152x; 1.0152x over previous
import functools

import jax
import jax.numpy as jnp
import numpy as np
from jax.experimental import pallas as pl
from jax.experimental.pallas import tpu as pltpu

_E = 64
_K = 2
_BLK = 1024
_SUB = 256


@functools.lru_cache(maxsize=None)
def _tril_const(blk):
    return jnp.asarray(np.tril(np.ones((blk, blk), dtype=np.float32))
                       .astype(jnp.bfloat16))


def _pass1(n_tokens, x_ref, wg_ref, tril_ref,
           logits_ref, gates_ref, wide_ref, laux_ref,
           run0, run1, me_acc):
    i = pl.program_id(0)
    rows = pl.ds(i * _BLK, _BLK)

    @pl.when(i == 0)
    def _init():
        run0[...] = jnp.zeros_like(run0)
        run1[...] = jnp.zeros_like(run1)
        me_acc[...] = jnp.zeros_like(me_acc)

    logits = jnp.dot(x_ref[...], wg_ref[...],
                     preferred_element_type=jnp.float32)
    logits_ref[...] = logits

    mx = jnp.max(logits, axis=1, keepdims=True)
    ex = jnp.exp(logits - mx)
    scores = ex / jnp.sum(ex, axis=1, keepdims=True)
    me_acc[...] += jnp.sum(scores, axis=0, keepdims=True)

    iota = jax.lax.broadcasted_iota(jnp.int32, scores.shape, 1)
    v0 = jnp.max(scores, axis=1, keepdims=True)
    i0 = jnp.min(jnp.where(scores == v0, iota, _E), axis=1, keepdims=True)
    m0b = iota == i0
    masked = jnp.where(m0b, -jnp.inf, scores)
    v1 = jnp.max(masked, axis=1, keepdims=True)
    i1 = jnp.min(jnp.where(masked == v1, iota, _E), axis=1, keepdims=True)
    m0 = m0b.astype(jnp.float32)
    m1 = (iota == i1).astype(jnp.float32)

    mcat = jnp.concatenate([m0, m1], axis=1).astype(jnp.bfloat16)
    tril = tril_ref[...]
    base = jnp.concatenate([run0[...], run1[...]], axis=1)
    locp = []
    for s in range(_BLK // _SUB):
        pref_s = jnp.dot(tril, mcat[s * _SUB:(s + 1) * _SUB, :],
                         preferred_element_type=jnp.float32)
        full_s = pref_s + (base - 1.0)
        m0_s = m0[s * _SUB:(s + 1) * _SUB, :]
        m1_s = m1[s * _SUB:(s + 1) * _SUB, :]
        loc0_s = jnp.sum(full_s[:, :_E] * m0_s, axis=1, keepdims=True)
        loc1_s = jnp.sum(full_s[:, _E:] * m1_s, axis=1, keepdims=True)
        locp.append(jnp.concatenate([loc0_s, loc1_s], axis=1))
        base = base + pref_s[_SUB - 1:_SUB, :]

    wide_ref[rows, :] = jnp.concatenate(
        [jnp.concatenate(locp, axis=0).astype(jnp.int32), i0, i1], axis=1)
    den = jnp.maximum(v0 + v1, 1e-9)
    gates_ref[...] = jnp.concatenate([v0 / den, v1 / den], axis=1)

    run0[...] = base[:, :_E]
    run1[...] = base[:, _E:]

    @pl.when(i == pl.num_programs(0) - 1)
    def _epilogue():
        i1_all = wide_ref[:, 3:4]
        iota_all = jax.lax.broadcasted_iota(jnp.int32, (n_tokens, _E), 1)
        m1_all = (iota_all == i1_all).astype(jnp.float32)
        add = jnp.sum(m1_all * run0[...], axis=1, keepdims=True)
        wide_ref[:, 1:2] = wide_ref[:, 1:2] + add.astype(jnp.int32)
        scale = jnp.float32(_E) / jnp.float32(n_tokens * n_tokens)
        laux_ref[...] = (jnp.sum(me_acc[...] * run0[...]) * scale
                         ).reshape(1, 1)


def kernel(x, wg, num_shards):
    n, d = x.shape
    nb = n // _BLK

    logits, gates, wide, laux = pl.pallas_call(
        functools.partial(_pass1, n),
        grid=(nb,),
        in_specs=[
            pl.BlockSpec((_BLK, d), lambda i: (i, 0)),
            pl.BlockSpec((d, _E), lambda i: (0, 0)),
            pl.BlockSpec((_SUB, _SUB), lambda i: (0, 0)),
        ],
        out_specs=[
            pl.BlockSpec((_BLK, _E), lambda i: (i, 0)),
            pl.BlockSpec((_BLK, _K), lambda i: (i, 0)),
            pl.BlockSpec((n, 2 * _K), lambda i: (0, 0)),
            pl.BlockSpec((1, 1), lambda i: (0, 0)),
        ],
        out_shape=[
            jax.ShapeDtypeStruct((n, _E), jnp.float32),
            jax.ShapeDtypeStruct((n, _K), jnp.float32),
            jax.ShapeDtypeStruct((n, 2 * _K), jnp.int32),
            jax.ShapeDtypeStruct((1, 1), jnp.float32),
        ],
        scratch_shapes=[
            pltpu.VMEM((1, _E), jnp.float32),
            pltpu.VMEM((1, _E), jnp.float32),
            pltpu.VMEM((1, _E), jnp.float32),
        ],
    )(x, wg, _tril_const(_SUB))

    locations = wide[:, :_K]
    topk_idx = wide[:, _K:]
    l_aux = laux.reshape(())
    alignment = jnp.asarray(num_shards, dtype=jnp.int32) * 1
    capacity = _K * ((n + _E - 1) // _E)
    cap_arr = (((capacity + alignment - 1) // alignment) * alignment
               ).astype(jnp.int32)
    return (logits, l_aux, topk_idx, locations, gates, cap_arr)


# speedup vs baseline: 1.1015x; 1.0850x over previous
import functools

import jax
import jax.numpy as jnp
import numpy as np
from jax.experimental import pallas as pl
from jax.experimental.pallas import tpu as pltpu

_E = 64
_K = 2
_BLK = 1024
_SUB = 256


@functools.lru_cache(maxsize=None)
def _tril_const(blk):
    return jnp.asarray(np.tril(np.ones((blk, blk), dtype=np.float32))
                       .astype(jnp.bfloat16))


def _pass1(n_tokens, x_ref, wg_ref, tril_ref,
           logits_ref, gates_ref, wide_ref, laux_ref,
           run0, run1, me_acc):
    i = pl.program_id(0)
    rows = pl.ds(i * _BLK, _BLK)

    @pl.when(i == 0)
    def _init():
        run0[...] = jnp.zeros_like(run0)
        run1[...] = jnp.zeros_like(run1)
        me_acc[...] = jnp.zeros_like(me_acc)

    logits = jnp.dot(x_ref[...], wg_ref[...],
                     preferred_element_type=jnp.float32)
    logits_ref[...] = logits

    mx = jnp.max(logits, axis=1, keepdims=True)
    ex = jnp.exp(logits - mx)
    scores = ex / jnp.sum(ex, axis=1, keepdims=True)
    me_acc[...] += jnp.sum(scores, axis=0, keepdims=True)

    iota = jax.lax.broadcasted_iota(jnp.int32, scores.shape, 1)
    v0 = jnp.max(scores, axis=1, keepdims=True)
    i0 = jnp.min(jnp.where(scores == v0, iota, _E), axis=1, keepdims=True)
    m0b = iota == i0
    masked = jnp.where(m0b, -jnp.inf, scores)
    v1 = jnp.max(masked, axis=1, keepdims=True)
    i1 = jnp.min(jnp.where(masked == v1, iota, _E), axis=1, keepdims=True)
    m0 = m0b.astype(jnp.float32)
    m1 = (iota == i1).astype(jnp.float32)

    mcat32 = jnp.concatenate([m0, m1], axis=1)
    mcat = mcat32.astype(jnp.bfloat16)
    tril = tril_ref[...]
    base = jnp.concatenate([run0[...], run1[...]], axis=1)
    full = []
    for s in range(_BLK // _SUB):
        pref_s = jnp.dot(tril, mcat[s * _SUB:(s + 1) * _SUB, :],
                         preferred_element_type=jnp.float32)
        full.append(pref_s + (base - 1.0))
        base = base + pref_s[_SUB - 1:_SUB, :]
    full = jnp.concatenate(full, axis=0)

    er = jax.lax.broadcasted_iota(jnp.int32, (2 * _E, _K), 0)
    ec = jax.lax.broadcasted_iota(jnp.int32, (2 * _E, _K), 1)
    sel = ((er < _E) == (ec == 0)).astype(jnp.float32)
    loc_pair = jnp.dot(full * mcat32, sel,
                       preferred_element_type=jnp.float32)

    wide_ref[rows, :] = jnp.concatenate(
        [loc_pair.astype(jnp.int32), i0, i1], axis=1)
    den = jnp.maximum(v0 + v1, 1e-9)
    gates_ref[...] = jnp.concatenate([v0 / den, v1 / den], axis=1)

    run0[...] = base[:, :_E]
    run1[...] = base[:, _E:]

    @pl.when(i == pl.num_programs(0) - 1)
    def _epilogue():
        i1_all = wide_ref[:, 3:4]
        iota_all = jax.lax.broadcasted_iota(jnp.int32, (n_tokens, _E), 1)
        m1_all = (iota_all == i1_all).astype(jnp.float32)
        ones_col = jnp.ones((_E, 1), dtype=jnp.float32)
        add = jnp.dot(m1_all * run0[...], ones_col,
                      preferred_element_type=jnp.float32)
        wide_ref[:, 1:2] = wide_ref[:, 1:2] + add.astype(jnp.int32)
        scale = jnp.float32(_E) / jnp.float32(n_tokens * n_tokens)
        laux_ref[...] = (jnp.sum(me_acc[...] * run0[...]) * scale
                         ).reshape(1, 1)


def kernel(x, wg, num_shards):
    n, d = x.shape
    nb = n // _BLK

    logits, gates, wide, laux = pl.pallas_call(
        functools.partial(_pass1, n),
        grid=(nb,),
        in_specs=[
            pl.BlockSpec((_BLK, d), lambda i: (i, 0)),
            pl.BlockSpec((d, _E), lambda i: (0, 0)),
            pl.BlockSpec((_SUB, _SUB), lambda i: (0, 0)),
        ],
        out_specs=[
            pl.BlockSpec((_BLK, _E), lambda i: (i, 0)),
            pl.BlockSpec((_BLK, _K), lambda i: (i, 0)),
            pl.BlockSpec((n, 2 * _K), lambda i: (0, 0)),
            pl.BlockSpec((1, 1), lambda i: (0, 0)),
        ],
        out_shape=[
            jax.ShapeDtypeStruct((n, _E), jnp.float32),
            jax.ShapeDtypeStruct((n, _K), jnp.float32),
            jax.ShapeDtypeStruct((n, 2 * _K), jnp.int32),
            jax.ShapeDtypeStruct((1, 1), jnp.float32),
        ],
        scratch_shapes=[
            pltpu.VMEM((1, _E), jnp.float32),
            pltpu.VMEM((1, _E), jnp.float32),
            pltpu.VMEM((1, _E), jnp.float32),
        ],
    )(x, wg, _tril_const(_SUB))

    locations = wide[:, :_K]
    topk_idx = wide[:, _K:]
    l_aux = laux.reshape(())
    alignment = jnp.asarray(num_shards, dtype=jnp.int32) * 1
    capacity = _K * ((n + _E - 1) // _E)
    cap_arr = (((capacity + alignment - 1) // alignment) * alignment
               ).astype(jnp.int32)
    return (logits, l_aux, topk_idx, locations, gates, cap_arr)
